# baseline (speedup 1.0000x reference)
import jax
import jax.numpy as jnp
from jax.experimental import pallas as pl
from jax.experimental.pallas import tpu as pltpu

IN_DIM = 50000
HIDDEN = 512
OUT_DIM = 50000
BATCH = 1024

KB = 2000
KSTEPS = IN_DIM // KB
NB = 1024
NSTEPS = (OUT_DIM + NB - 1) // NB

W1S = 256.0
W2S = 64.0


def _mlp_kernel(xt_ref, w1t_ref, b1_ref, w2_ref, b2_ref, ot_ref, acc_ref, h_ref):
    k = pl.program_id(0)
    f8 = jnp.float8_e4m3fn

    @pl.when(k == 0)
    def _init():
        acc_ref[...] = jnp.zeros_like(acc_ref)

    @pl.when(k < KSTEPS)
    def _fc1():
        xb = xt_ref[...].astype(f8)
        wb = (w1t_ref[...] * W1S).astype(f8)
        acc_ref[...] += jax.lax.dot_general(
            xb, wb, (((0,), (0,)), ((), ())), preferred_element_type=jnp.float32
        )

    @pl.when(k == KSTEPS - 1)
    def _fc1_epilogue():
        b1row = jnp.reshape(b1_ref[...], (1, HIDDEN))
        h_ref[...] = jnp.maximum(acc_ref[...] * (1.0 / W1S) + b1row, 0.0)

    @pl.when(k >= KSTEPS)
    def _fc2():
        hb = h_ref[...].astype(f8)
        wb = (w2_ref[...] * W2S).astype(f8)
        acc = jax.lax.dot_general(
            wb, hb, (((1,), (1,)), ((), ())), preferred_element_type=jnp.float32
        )
        bcol = jnp.reshape(b2_ref[...], (NB, 1))
        y = acc * (1.0 / W2S) + bcol
        ot_ref[...] = 0.5 * jnp.tanh(0.5 * y) + 0.5


def kernel(x, W1, b1, W2, b2):
    xt = x.T
    w1t = W1.T
    def a_idx(k):
        ka = jnp.minimum(k, KSTEPS - 1)
        return (ka, 0)

    def w2_idx(k):
        return (jnp.maximum(k - KSTEPS, 0), 0)

    def b2_idx(k):
        return (jnp.maximum(k - KSTEPS, 0),)

    def ot_idx(k):
        return (jnp.maximum(k - KSTEPS, 0), 0)

    ot = pl.pallas_call(
        _mlp_kernel,
        grid=(KSTEPS + NSTEPS,),
        in_specs=[
            pl.BlockSpec((KB, BATCH), a_idx),
            pl.BlockSpec((KB, HIDDEN), a_idx),
            pl.BlockSpec((HIDDEN,), lambda k: (0,)),
            pl.BlockSpec((NB, HIDDEN), w2_idx),
            pl.BlockSpec((NB,), b2_idx),
        ],
        out_specs=pl.BlockSpec((NB, BATCH), ot_idx),
        out_shape=jax.ShapeDtypeStruct((OUT_DIM, BATCH), jnp.float32),
        scratch_shapes=[
            pltpu.VMEM((BATCH, HIDDEN), jnp.float32),
            pltpu.VMEM((BATCH, HIDDEN), jnp.float32),
        ],
    )(xt, w1t, b1, W2, b2)
    return ot.T


# speedup vs baseline: 1.0741x; 1.0741x over previous
import jax
import jax.numpy as jnp
from jax.experimental import pallas as pl
from jax.experimental.pallas import tpu as pltpu

IN_DIM = 50000
HIDDEN = 512
OUT_DIM = 50000
BATCH = 1024

KB = 2000
KSTEPS = IN_DIM // KB
NB = 2048
NSTEPS = (OUT_DIM + NB - 1) // NB

W1S = 256.0
W2S = 64.0


def _mlp_kernel(xt_ref, w1t_ref, b1_ref, w2_ref, b2_ref, ot_ref, acc_ref, h_ref):
    k = pl.program_id(0)
    f8 = jnp.float8_e4m3fn

    @pl.when(k == 0)
    def _init():
        acc_ref[...] = jnp.zeros_like(acc_ref)

    @pl.when(k < KSTEPS)
    def _fc1():
        xb = xt_ref[...].astype(f8)
        wb = (w1t_ref[...] * W1S).astype(f8)
        acc_ref[...] += jax.lax.dot_general(
            xb, wb, (((0,), (0,)), ((), ())), preferred_element_type=jnp.float32
        )

    @pl.when(k == KSTEPS - 1)
    def _fc1_epilogue():
        b1row = jnp.reshape(b1_ref[...], (1, HIDDEN))
        h_ref[...] = jnp.maximum(acc_ref[...] * (1.0 / W1S) + b1row, 0.0)

    @pl.when(k >= KSTEPS)
    def _fc2():
        hb = h_ref[...].astype(f8)
        wb = (w2_ref[...] * W2S).astype(f8)
        acc = jax.lax.dot_general(
            wb, hb, (((1,), (1,)), ((), ())), preferred_element_type=jnp.float32
        )
        bcol = jnp.reshape(b2_ref[...], (NB, 1))
        y = acc * (0.5 / W2S) + bcol * 0.5
        t = jnp.tanh(y.astype(jnp.bfloat16))
        ot_ref[...] = (t * jnp.bfloat16(0.5) + jnp.bfloat16(0.5)).astype(
            jnp.float32
        )


def kernel(x, W1, b1, W2, b2):
    xt = x.T
    w1t = W1.T
    def a_idx(k):
        ka = jnp.minimum(k, KSTEPS - 1)
        return (ka, 0)

    def w2_idx(k):
        return (jnp.maximum(k - KSTEPS, 0), 0)

    def b2_idx(k):
        return (jnp.maximum(k - KSTEPS, 0),)

    def ot_idx(k):
        return (jnp.maximum(k - KSTEPS, 0), 0)

    ot = pl.pallas_call(
        _mlp_kernel,
        grid=(KSTEPS + NSTEPS,),
        in_specs=[
            pl.BlockSpec((KB, BATCH), a_idx),
            pl.BlockSpec((KB, HIDDEN), a_idx),
            pl.BlockSpec((HIDDEN,), lambda k: (0,)),
            pl.BlockSpec((NB, HIDDEN), w2_idx),
            pl.BlockSpec((NB,), b2_idx),
        ],
        out_specs=pl.BlockSpec((NB, BATCH), ot_idx),
        out_shape=jax.ShapeDtypeStruct((OUT_DIM, BATCH), jnp.float32),
        scratch_shapes=[
            pltpu.VMEM((BATCH, HIDDEN), jnp.float32),
            pltpu.VMEM((BATCH, HIDDEN), jnp.float32),
        ],
    )(xt, w1t, b1, W2, b2)
    return ot.T
